# baseline (speedup 1.0000x reference)
import functools

import jax
import jax.numpy as jnp
from jax import lax
from jax.experimental import pallas as pl
from jax.experimental.pallas import tpu as pltpu
from jax.experimental.pallas import tpu_sc as plsc

NC = 2
NS = 16
CHUNK = 128
CORE0_FRAC = 0.65


def _mesh():
    return plsc.VectorSubcoreMesh(core_axis_name="c", subcore_axis_name="s")


def _chunk_split(p):
    a = min(p - 8, max(8, int(round(p * CORE0_FRAC / 4)) * 4))
    return a, p - a


def _make_deg_kernel(n, n_acc, p):
    rows_per_tile = n_acc // NS
    last = n - (NS - 1) * rows_per_tile
    a, b = _chunk_split(p)

    @functools.partial(
        pl.kernel,
        out_type=jax.ShapeDtypeStruct((NC, n), jnp.float32),
        mesh=_mesh(),
        scratch_types=[
            pltpu.VMEM((max(a, b), CHUNK), jnp.int32),
            pltpu.VMEM((CHUNK,), jnp.float32),
            pltpu.VMEM((CHUNK,), jnp.float32),
            pltpu.VMEM_SHARED((n_acc,), jnp.float32),
            pltpu.SemaphoreType.DMA,
        ],
        compiler_params=pltpu.CompilerParams(use_tc_tiling_on_sc=False),
    )
    def deg_kernel(dst_hbm, out_hbm, idx_v, ones_v, zeros_v, acc_sp, sem):
        c = lax.axis_index("c")
        s = lax.axis_index("s")
        for i in range(CHUNK // 16):
            ones_v[pl.ds(16 * i, 16)] = jnp.ones((16,), jnp.float32)
            zeros_v[pl.ds(16 * i, 16)] = jnp.zeros((16,), jnp.float32)
        base = s * rows_per_tile
        for k in range(rows_per_tile // CHUNK):
            pltpu.sync_copy(zeros_v, acc_sp.at[pl.ds(base + k * CHUNK, CHUNK)])

        def scatter_loop(cnt):
            depth = 8

            def wait_s():
                pltpu.make_async_copy(out_hbm.at[0, pl.ds(0, CHUNK)],
                                      ones_v, sem).wait()

            def body(j, carry):
                pltpu.async_copy(ones_v, acc_sp.at[idx_v.at[j]], sem,
                                 add=True)

                @pl.when(j >= depth)
                def _():
                    wait_s()

                return carry

            lax.fori_loop(0, cnt, body, 0)
            for _ in range(depth):
                wait_s()

        @pl.when(c == 0)
        def _():
            pltpu.sync_copy(dst_hbm.at[pl.ds(s * a, a)],
                            idx_v.at[pl.ds(0, a)])

        @pl.when(c == 1)
        def _():
            pltpu.sync_copy(dst_hbm.at[pl.ds(NS * a + s * b, b)],
                            idx_v.at[pl.ds(0, b)])

        plsc.subcore_barrier()

        @pl.when(c == 0)
        def _():
            scatter_loop(a)

        @pl.when(c == 1)
        def _():
            scatter_loop(b)

        plsc.subcore_barrier()

        @pl.when(s < NS - 1)
        def _():
            pltpu.sync_copy(acc_sp.at[pl.ds(base, rows_per_tile)],
                            out_hbm.at[c, pl.ds(base, rows_per_tile)])

        @pl.when(s == NS - 1)
        def _():
            pltpu.sync_copy(acc_sp.at[pl.ds(base, last)],
                            out_hbm.at[c, pl.ds(base, last)])

    return deg_kernel


def _make_agg_kernel(n, n_acc, p, width):
    rows_per_tile = n_acc // NS
    last = n - (NS - 1) * rows_per_tile
    a, b = _chunk_split(p)

    @functools.partial(
        pl.kernel,
        out_type=jax.ShapeDtypeStruct((NC, n, width), jnp.float32),
        mesh=_mesh(),
        scratch_types=[
            pltpu.VMEM((max(a, b), CHUNK), jnp.int32),
            pltpu.VMEM((max(a, b), CHUNK), jnp.int32),
            pltpu.VMEM((4, CHUNK, width), jnp.float32),
            pltpu.VMEM_SHARED((n_acc, width), jnp.float32),
            pltpu.SemaphoreType.DMA,
            pltpu.SemaphoreType.DMA,
        ],
        compiler_params=pltpu.CompilerParams(use_tc_tiling_on_sc=False),
    )
    def agg_kernel(g_hbm, src_hbm, dst_hbm, out_hbm, si_v, di_v, msg_v,
                   acc_sp, sem_g, sem_s):
        c = lax.axis_index("c")
        s = lax.axis_index("s")
        base = s * rows_per_tile
        @pl.when(s < NS - 1)
        def _():
            pltpu.sync_copy(g_hbm.at[pl.ds(base, rows_per_tile)],
                            acc_sp.at[pl.ds(base, rows_per_tile)])

        @pl.when(s == NS - 1)
        def _():
            pltpu.sync_copy(g_hbm.at[pl.ds(base, last)],
                            acc_sp.at[pl.ds(base, last)])

        @pl.when(c == 0)
        def _():
            pltpu.sync_copy(src_hbm.at[pl.ds(s * a, a)],
                            si_v.at[pl.ds(0, a)])
            pltpu.sync_copy(dst_hbm.at[pl.ds(s * a, a)],
                            di_v.at[pl.ds(0, a)])

        @pl.when(c == 1)
        def _():
            pltpu.sync_copy(src_hbm.at[pl.ds(NS * a + s * b, b)],
                            si_v.at[pl.ds(0, b)])
            pltpu.sync_copy(dst_hbm.at[pl.ds(NS * a + s * b, b)],
                            di_v.at[pl.ds(0, b)])

        plsc.subcore_barrier()

        def edge_loop(cnt):
            def fire_g(j, slot):
                pltpu.async_copy(g_hbm.at[si_v.at[j]], msg_v.at[slot],
                                 sem_g)

            def fire_s(j, slot):
                pltpu.async_copy(msg_v.at[slot], acc_sp.at[di_v.at[j]],
                                 sem_s, add=True)

            def wait_g():
                pltpu.make_async_copy(g_hbm.at[pl.ds(0, CHUNK)],
                                      msg_v.at[0], sem_g).wait()

            def wait_s():
                pltpu.make_async_copy(g_hbm.at[pl.ds(0, CHUNK)],
                                      msg_v.at[0], sem_s).wait()

            fire_g(0, 0)
            fire_g(1, 1)

            def body(i, carry):
                for b in range(4):
                    j = 4 * i + b
                    wait_g()
                    fire_s(j, b)

                    @pl.when(j >= 2)
                    def _():
                        wait_s()

                    @pl.when(j + 2 < cnt)
                    def _():
                        fire_g(j + 2, (b + 2) % 4)

                return carry

            lax.fori_loop(0, cnt // 4, body, 0)
            wait_s()
            wait_s()

        @pl.when(c == 0)
        def _():
            edge_loop(a)

        @pl.when(c == 1)
        def _():
            edge_loop(b)

        plsc.subcore_barrier()

        @pl.when(s < NS - 1)
        def _():
            pltpu.sync_copy(acc_sp.at[pl.ds(base, rows_per_tile)],
                            out_hbm.at[c, pl.ds(base, rows_per_tile)])

        @pl.when(s == NS - 1)
        def _():
            pltpu.sync_copy(acc_sp.at[pl.ds(base, last)],
                            out_hbm.at[c, pl.ds(base, last)])

    return agg_kernel


def _tc_matmul(x3, W1):
    n8, _, d = x3.shape
    h = W1.shape[1]

    def body(x_ref, w_ref, h_ref):
        w = w_ref[...]
        for k in range(8):
            h_ref[:, h * k:h * (k + 1)] = jnp.dot(
                x_ref[:, k, :], w, preferred_element_type=jnp.float32)

    return pl.pallas_call(
        body,
        out_shape=jax.ShapeDtypeStruct((n8, 8 * h), jnp.float32),
    )(x3, W1)


def _tc_scale(h_pack, d3):
    n8, hw = h_pack.shape
    h = hw // 8

    def body(h_ref, d_ref, g_ref, s_ref):
        dd = d_ref[...]
        hh = h_ref[...]
        for k in range(8):
            sck = lax.rsqrt(1.0 + dd[0, :, k] + dd[1, :, k])[:, None]
            g_ref[:, h * k:h * (k + 1)] = sck * hh[:, h * k:h * (k + 1)]
            s_ref[:, h * k:h * (k + 1)] = jnp.broadcast_to(sck, (n8, h))

    return pl.pallas_call(
        body,
        out_shape=(
            jax.ShapeDtypeStruct((n8, hw), jnp.float32),
            jax.ShapeDtypeStruct((n8, hw), jnp.float32),
        ),
    )(h_pack, d3)


def _tc_mid(p, g1, s_pack, b1p):
    def body(p_ref, g_ref, s_ref, b_ref, u_ref):
        sc = s_ref[...]
        agg = p_ref[0] + p_ref[1] - g_ref[...]
        u_ref[...] = sc * jnp.maximum(sc * agg + b_ref[...], 0.0)

    return pl.pallas_call(
        body,
        out_shape=jax.ShapeDtypeStruct(g1.shape, jnp.float32),
    )(p, g1, s_pack, b1p)


def _tc_final(q, u, s_pack, W2p, b2p, c):
    n8, hw = u.shape
    cw = 8 * c

    def body(q_ref, u_ref, s_ref, w_ref, b_ref, o_ref):
        t = s_ref[...] * (q_ref[0] + q_ref[1] - u_ref[...])
        z = jnp.dot(t, w_ref[...], preferred_element_type=jnp.float32)
        z = z + b_ref[...]
        zl = jnp.concatenate([z[:, 1:], z[:, :1]], axis=1)
        zr = jnp.concatenate([z[:, cw - 1:], z[:, :cw - 1]], axis=1)
        lane = lax.broadcasted_iota(jnp.int32, (n8, cw), 1)
        partner = jnp.where(lane % 2 == 0, zl, zr)
        m = jnp.maximum(z, partner)
        lse = m + jnp.log(jnp.exp(z - m) + jnp.exp(partner - m))
        o_ref[...] = z - lse

    return pl.pallas_call(
        body,
        out_shape=jax.ShapeDtypeStruct((n8, cw), jnp.float32),
    )(q, u, s_pack, W2p, b2p)


def kernel(x, edge_index, W1, b1, W2, b2):
    n, d = x.shape
    h = W1.shape[1]
    c = W2.shape[1]
    e = edge_index.shape[1]
    n8 = n // 8

    n_acc = ((n + NS * 8) + NS * 8 - 1) // (NS * 8) * (NS * 8)
    p = (e + NS * CHUNK - 1) // (NS * CHUNK)
    p = p + (p % 2)
    e_pad = NS * CHUNK * p

    src = jnp.concatenate(
        [edge_index[0], jnp.zeros((e_pad - e,), jnp.int32)]).reshape(
            NS * p, CHUNK)
    dst = jnp.concatenate(
        [edge_index[1], jnp.full((e_pad - e,), n, jnp.int32)]).reshape(
            NS * p, CHUNK)

    deg_kernel = _make_deg_kernel(n, n_acc, p)
    agg_kernel = _make_agg_kernel(n, n_acc, p, h)

    x3 = x.reshape(n8, 8, d)
    b1p = jnp.tile(b1, 8).reshape(1, 8 * h)
    W2p = jnp.kron(jnp.eye(8, dtype=W2.dtype), W2)
    b2p = jnp.tile(b2, 8).reshape(1, 8 * c)

    h_pack = _tc_matmul(x3, W1)
    degp = deg_kernel(dst)
    g1, s_pack = _tc_scale(h_pack, degp.reshape(NC, n8, 8))
    pagg = agg_kernel(g1.reshape(n, h), src, dst)
    u = _tc_mid(pagg.reshape(NC, n8, 8 * h), g1, s_pack, b1p)
    q = agg_kernel(u.reshape(n, h), src, dst)
    out = _tc_final(q.reshape(NC, n8, 8 * h), u, s_pack, W2p, b2p, c)
    return out.reshape(n, c)


# speedup vs baseline: 1.0990x; 1.0990x over previous
import functools

import jax
import jax.numpy as jnp
from jax import lax
from jax.experimental import pallas as pl
from jax.experimental.pallas import tpu as pltpu
from jax.experimental.pallas import tpu_sc as plsc

NC = 2
NS = 16
CHUNK = 128
CORE0_FRAC = 0.65


def _mesh():
    return plsc.VectorSubcoreMesh(core_axis_name="c", subcore_axis_name="s")


def _chunk_split(p):
    a = min(p - 8, max(8, int(round(p * CORE0_FRAC / 4)) * 4))
    return a, p - a


def _make_deg_kernel(n, n_acc, p):
    rows_per_tile = n_acc // NS
    last = n - (NS - 1) * rows_per_tile
    a, b = _chunk_split(p)

    @functools.partial(
        pl.kernel,
        out_type=jax.ShapeDtypeStruct((NC, n), jnp.float32),
        mesh=_mesh(),
        scratch_types=[
            pltpu.VMEM((max(a, b), CHUNK), jnp.int32),
            pltpu.VMEM((CHUNK,), jnp.float32),
            pltpu.VMEM((CHUNK,), jnp.float32),
            pltpu.VMEM_SHARED((n_acc,), jnp.float32),
            pltpu.SemaphoreType.DMA,
        ],
        compiler_params=pltpu.CompilerParams(use_tc_tiling_on_sc=False),
    )
    def deg_kernel(dst_hbm, out_hbm, idx_v, ones_v, zeros_v, acc_sp, sem):
        c = lax.axis_index("c")
        s = lax.axis_index("s")
        for i in range(CHUNK // 16):
            ones_v[pl.ds(16 * i, 16)] = jnp.ones((16,), jnp.float32)
            zeros_v[pl.ds(16 * i, 16)] = jnp.zeros((16,), jnp.float32)
        base = s * rows_per_tile
        for k in range(rows_per_tile // CHUNK):
            pltpu.sync_copy(zeros_v, acc_sp.at[pl.ds(base + k * CHUNK, CHUNK)])

        def scatter_loop(cnt):
            depth = 8

            def wait_s():
                pltpu.make_async_copy(out_hbm.at[0, pl.ds(0, CHUNK)],
                                      ones_v, sem).wait()

            def body(j, carry):
                pltpu.async_copy(ones_v, acc_sp.at[idx_v.at[j]], sem,
                                 add=True)

                @pl.when(j >= depth)
                def _():
                    wait_s()

                return carry

            lax.fori_loop(0, cnt, body, 0)
            for _ in range(depth):
                wait_s()

        @pl.when(c == 0)
        def _():
            pltpu.sync_copy(dst_hbm.at[pl.ds(s * a, a)],
                            idx_v.at[pl.ds(0, a)])

        @pl.when(c == 1)
        def _():
            pltpu.sync_copy(dst_hbm.at[pl.ds(NS * a + s * b, b)],
                            idx_v.at[pl.ds(0, b)])

        plsc.subcore_barrier()

        @pl.when(c == 0)
        def _():
            scatter_loop(a)

        @pl.when(c == 1)
        def _():
            scatter_loop(b)

        plsc.subcore_barrier()

        @pl.when(s < NS - 1)
        def _():
            pltpu.sync_copy(acc_sp.at[pl.ds(base, rows_per_tile)],
                            out_hbm.at[c, pl.ds(base, rows_per_tile)])

        @pl.when(s == NS - 1)
        def _():
            pltpu.sync_copy(acc_sp.at[pl.ds(base, last)],
                            out_hbm.at[c, pl.ds(base, last)])

    return deg_kernel


def _make_agg_kernel(n, n_acc, p, width):
    rows_per_tile = n_acc // NS
    last = n - (NS - 1) * rows_per_tile
    a, b = _chunk_split(p)

    @functools.partial(
        pl.kernel,
        out_type=jax.ShapeDtypeStruct((NC, n, width), jnp.float32),
        mesh=_mesh(),
        scratch_types=[
            pltpu.VMEM((max(a, b), CHUNK), jnp.int32),
            pltpu.VMEM((max(a, b), CHUNK), jnp.int32),
            pltpu.VMEM((4, CHUNK, width), jnp.float32),
            pltpu.VMEM_SHARED((n_acc, width), jnp.float32),
            pltpu.SemaphoreType.DMA,
            pltpu.SemaphoreType.DMA,
        ],
        compiler_params=pltpu.CompilerParams(use_tc_tiling_on_sc=False),
    )
    def agg_kernel(g_hbm, src_hbm, dst_hbm, out_hbm, si_v, di_v, msg_v,
                   acc_sp, sem_g, sem_s):
        c = lax.axis_index("c")
        s = lax.axis_index("s")
        base = s * rows_per_tile
        @pl.when(s < NS - 1)
        def _():
            pltpu.sync_copy(g_hbm.at[pl.ds(base, rows_per_tile)],
                            acc_sp.at[pl.ds(base, rows_per_tile)])

        @pl.when(s == NS - 1)
        def _():
            pltpu.sync_copy(g_hbm.at[pl.ds(base, last)],
                            acc_sp.at[pl.ds(base, last)])

        @pl.when(c == 0)
        def _():
            pltpu.sync_copy(src_hbm.at[pl.ds(s * a, a)],
                            si_v.at[pl.ds(0, a)])
            pltpu.sync_copy(dst_hbm.at[pl.ds(s * a, a)],
                            di_v.at[pl.ds(0, a)])

        @pl.when(c == 1)
        def _():
            pltpu.sync_copy(src_hbm.at[pl.ds(NS * a + s * b, b)],
                            si_v.at[pl.ds(0, b)])
            pltpu.sync_copy(dst_hbm.at[pl.ds(NS * a + s * b, b)],
                            di_v.at[pl.ds(0, b)])

        plsc.subcore_barrier()

        def edge_loop(cnt):
            half = cnt // 2

            def fire_g(j, slot):
                pltpu.async_copy(g_hbm.at[si_v.at[j]], msg_v.at[slot],
                                 sem_g)

            def wait_g():
                pltpu.make_async_copy(g_hbm.at[pl.ds(0, CHUNK)],
                                      msg_v.at[0], sem_g).wait()

            fire_g(0, 0)
            fire_g(1, 1)

            def body(i, carry):
                wait_g()
                pltpu.sync_copy(msg_v.at[0], acc_sp.at[di_v.at[2 * i]],
                                add=True)

                @pl.when(i < half - 1)
                def _():
                    fire_g(2 * i + 2, 0)

                wait_g()
                pltpu.sync_copy(msg_v.at[1], acc_sp.at[di_v.at[2 * i + 1]],
                                add=True)

                @pl.when(i < half - 1)
                def _():
                    fire_g(2 * i + 3, 1)

                return carry

            lax.fori_loop(0, half, body, 0)

        @pl.when(c == 0)
        def _():
            edge_loop(a)

        @pl.when(c == 1)
        def _():
            edge_loop(b)

        plsc.subcore_barrier()

        @pl.when(s < NS - 1)
        def _():
            pltpu.sync_copy(acc_sp.at[pl.ds(base, rows_per_tile)],
                            out_hbm.at[c, pl.ds(base, rows_per_tile)])

        @pl.when(s == NS - 1)
        def _():
            pltpu.sync_copy(acc_sp.at[pl.ds(base, last)],
                            out_hbm.at[c, pl.ds(base, last)])

    return agg_kernel


def _tc_matmul(x3, W1):
    n8, _, d = x3.shape
    h = W1.shape[1]

    def body(x_ref, w_ref, h_ref):
        w = w_ref[...]
        for k in range(8):
            h_ref[:, h * k:h * (k + 1)] = jnp.dot(
                x_ref[:, k, :], w, preferred_element_type=jnp.float32)

    return pl.pallas_call(
        body,
        out_shape=jax.ShapeDtypeStruct((n8, 8 * h), jnp.float32),
    )(x3, W1)


def _tc_scale(h_pack, d3):
    n8, hw = h_pack.shape
    h = hw // 8

    def body(h_ref, d_ref, g_ref, s_ref):
        dd = d_ref[...]
        hh = h_ref[...]
        for k in range(8):
            sck = lax.rsqrt(1.0 + dd[0, :, k] + dd[1, :, k])[:, None]
            g_ref[:, h * k:h * (k + 1)] = sck * hh[:, h * k:h * (k + 1)]
            s_ref[:, h * k:h * (k + 1)] = jnp.broadcast_to(sck, (n8, h))

    return pl.pallas_call(
        body,
        out_shape=(
            jax.ShapeDtypeStruct((n8, hw), jnp.float32),
            jax.ShapeDtypeStruct((n8, hw), jnp.float32),
        ),
    )(h_pack, d3)


def _tc_mid(p, g1, s_pack, b1p):
    def body(p_ref, g_ref, s_ref, b_ref, u_ref):
        sc = s_ref[...]
        agg = p_ref[0] + p_ref[1] - g_ref[...]
        u_ref[...] = sc * jnp.maximum(sc * agg + b_ref[...], 0.0)

    return pl.pallas_call(
        body,
        out_shape=jax.ShapeDtypeStruct(g1.shape, jnp.float32),
    )(p, g1, s_pack, b1p)


def _tc_final(q, u, s_pack, W2p, b2p, c):
    n8, hw = u.shape
    cw = 8 * c

    def body(q_ref, u_ref, s_ref, w_ref, b_ref, o_ref):
        t = s_ref[...] * (q_ref[0] + q_ref[1] - u_ref[...])
        z = jnp.dot(t, w_ref[...], preferred_element_type=jnp.float32)
        z = z + b_ref[...]
        zl = jnp.concatenate([z[:, 1:], z[:, :1]], axis=1)
        zr = jnp.concatenate([z[:, cw - 1:], z[:, :cw - 1]], axis=1)
        lane = lax.broadcasted_iota(jnp.int32, (n8, cw), 1)
        partner = jnp.where(lane % 2 == 0, zl, zr)
        m = jnp.maximum(z, partner)
        lse = m + jnp.log(jnp.exp(z - m) + jnp.exp(partner - m))
        o_ref[...] = z - lse

    return pl.pallas_call(
        body,
        out_shape=jax.ShapeDtypeStruct((n8, cw), jnp.float32),
    )(q, u, s_pack, W2p, b2p)


def kernel(x, edge_index, W1, b1, W2, b2):
    n, d = x.shape
    h = W1.shape[1]
    c = W2.shape[1]
    e = edge_index.shape[1]
    n8 = n // 8

    n_acc = ((n + NS * 8) + NS * 8 - 1) // (NS * 8) * (NS * 8)
    p = (e + NS * CHUNK - 1) // (NS * CHUNK)
    p = p + (p % 2)
    e_pad = NS * CHUNK * p

    src = jnp.concatenate(
        [edge_index[0], jnp.zeros((e_pad - e,), jnp.int32)]).reshape(
            NS * p, CHUNK)
    dst = jnp.concatenate(
        [edge_index[1], jnp.full((e_pad - e,), n, jnp.int32)]).reshape(
            NS * p, CHUNK)

    deg_kernel = _make_deg_kernel(n, n_acc, p)
    agg_kernel = _make_agg_kernel(n, n_acc, p, h)

    x3 = x.reshape(n8, 8, d)
    b1p = jnp.tile(b1, 8).reshape(1, 8 * h)
    W2p = jnp.kron(jnp.eye(8, dtype=W2.dtype), W2)
    b2p = jnp.tile(b2, 8).reshape(1, 8 * c)

    h_pack = _tc_matmul(x3, W1)
    degp = deg_kernel(dst)
    g1, s_pack = _tc_scale(h_pack, degp.reshape(NC, n8, 8))
    pagg = agg_kernel(g1.reshape(n, h), src, dst)
    u = _tc_mid(pagg.reshape(NC, n8, 8 * h), g1, s_pack, b1p)
    q = agg_kernel(u.reshape(n, h), src, dst)
    out = _tc_final(q.reshape(NC, n8, 8 * h), u, s_pack, W2p, b2p, c)
    return out.reshape(n, c)


# speedup vs baseline: 1.0992x; 1.0001x over previous
import functools

import jax
import jax.numpy as jnp
from jax import lax
from jax.experimental import pallas as pl
from jax.experimental.pallas import tpu as pltpu
from jax.experimental.pallas import tpu_sc as plsc

NC = 2
NS = 16
CHUNK = 128
CORE0_FRAC = 0.65


def _mesh():
    return plsc.VectorSubcoreMesh(core_axis_name="c", subcore_axis_name="s")


def _chunk_split(p):
    a = min(p - 8, max(8, int(round(p * CORE0_FRAC / 4)) * 4))
    return a, p - a


def _make_deg_kernel(n, n_acc, p):
    rows_per_tile = n_acc // NS
    last = n - (NS - 1) * rows_per_tile
    a, b = _chunk_split(p)

    @functools.partial(
        pl.kernel,
        out_type=jax.ShapeDtypeStruct((NC, n), jnp.float32),
        mesh=_mesh(),
        scratch_types=[
            pltpu.VMEM((max(a, b), CHUNK), jnp.int32),
            pltpu.VMEM((CHUNK,), jnp.float32),
            pltpu.VMEM((CHUNK,), jnp.float32),
            pltpu.VMEM_SHARED((n_acc,), jnp.float32),
            pltpu.SemaphoreType.DMA,
        ],
        compiler_params=pltpu.CompilerParams(use_tc_tiling_on_sc=False),
    )
    def deg_kernel(dst_hbm, out_hbm, idx_v, ones_v, zeros_v, acc_sp, sem):
        c = lax.axis_index("c")
        s = lax.axis_index("s")
        for i in range(CHUNK // 16):
            ones_v[pl.ds(16 * i, 16)] = jnp.ones((16,), jnp.float32)
            zeros_v[pl.ds(16 * i, 16)] = jnp.zeros((16,), jnp.float32)
        base = s * rows_per_tile
        for k in range(rows_per_tile // CHUNK):
            pltpu.sync_copy(zeros_v, acc_sp.at[pl.ds(base + k * CHUNK, CHUNK)])

        def scatter_loop(cnt):
            depth = 8

            def wait_s():
                pltpu.make_async_copy(out_hbm.at[0, pl.ds(0, CHUNK)],
                                      ones_v, sem).wait()

            def body(j, carry):
                pltpu.async_copy(ones_v, acc_sp.at[idx_v.at[j]], sem,
                                 add=True)

                @pl.when(j >= depth)
                def _():
                    wait_s()

                return carry

            lax.fori_loop(0, cnt, body, 0)
            for _ in range(depth):
                wait_s()

        @pl.when(c == 0)
        def _():
            pltpu.sync_copy(dst_hbm.at[pl.ds(s * a, a)],
                            idx_v.at[pl.ds(0, a)])

        @pl.when(c == 1)
        def _():
            pltpu.sync_copy(dst_hbm.at[pl.ds(NS * a + s * b, b)],
                            idx_v.at[pl.ds(0, b)])

        plsc.subcore_barrier()

        @pl.when(c == 0)
        def _():
            scatter_loop(a)

        @pl.when(c == 1)
        def _():
            scatter_loop(b)

        plsc.subcore_barrier()

        @pl.when(s < NS - 1)
        def _():
            pltpu.sync_copy(acc_sp.at[pl.ds(base, rows_per_tile)],
                            out_hbm.at[c, pl.ds(base, rows_per_tile)])

        @pl.when(s == NS - 1)
        def _():
            pltpu.sync_copy(acc_sp.at[pl.ds(base, last)],
                            out_hbm.at[c, pl.ds(base, last)])

    return deg_kernel


def _make_agg_kernel(n, n_acc, p, width):
    rows_per_tile = n_acc // NS
    last = n - (NS - 1) * rows_per_tile
    a, b = _chunk_split(p)

    @functools.partial(
        pl.kernel,
        out_type=jax.ShapeDtypeStruct((NC, n, width), jnp.float32),
        mesh=_mesh(),
        scratch_types=[
            pltpu.VMEM((max(a, b), CHUNK), jnp.int32),
            pltpu.VMEM((max(a, b), CHUNK), jnp.int32),
            pltpu.VMEM((4, CHUNK, width), jnp.float32),
            pltpu.VMEM_SHARED((n_acc, width), jnp.float32),
            pltpu.SemaphoreType.DMA,
            pltpu.SemaphoreType.DMA,
        ],
        compiler_params=pltpu.CompilerParams(use_tc_tiling_on_sc=False),
    )
    def agg_kernel(g_hbm, src_hbm, dst_hbm, out_hbm, si_v, di_v, msg_v,
                   acc_sp, sem_g, sem_s):
        c = lax.axis_index("c")
        s = lax.axis_index("s")
        base = s * rows_per_tile
        @pl.when(s < NS - 1)
        def _():
            pltpu.sync_copy(g_hbm.at[pl.ds(base, rows_per_tile)],
                            acc_sp.at[pl.ds(base, rows_per_tile)])

        @pl.when(s == NS - 1)
        def _():
            pltpu.sync_copy(g_hbm.at[pl.ds(base, last)],
                            acc_sp.at[pl.ds(base, last)])

        @pl.when(c == 0)
        def _():
            pltpu.sync_copy(src_hbm.at[pl.ds(s * a, a)],
                            si_v.at[pl.ds(0, a)])
            pltpu.sync_copy(dst_hbm.at[pl.ds(s * a, a)],
                            di_v.at[pl.ds(0, a)])

        @pl.when(c == 1)
        def _():
            pltpu.sync_copy(src_hbm.at[pl.ds(NS * a + s * b, b)],
                            si_v.at[pl.ds(0, b)])
            pltpu.sync_copy(dst_hbm.at[pl.ds(NS * a + s * b, b)],
                            di_v.at[pl.ds(0, b)])

        plsc.subcore_barrier()

        def edge_loop(cnt):
            half = cnt // 2

            def fire_g(j, slot):
                pltpu.async_copy(g_hbm.at[si_v.at[j]], msg_v.at[slot],
                                 sem_g)

            def wait_g():
                pltpu.make_async_copy(g_hbm.at[pl.ds(0, CHUNK)],
                                      msg_v.at[0], sem_g).wait()

            fire_g(0, 0)
            fire_g(1, 1)

            def body(i, carry):
                wait_g()
                pltpu.sync_copy(msg_v.at[0], acc_sp.at[di_v.at[2 * i]],
                                add=True)

                @pl.when(i < half - 1)
                def _():
                    fire_g(2 * i + 2, 0)

                wait_g()
                pltpu.sync_copy(msg_v.at[1], acc_sp.at[di_v.at[2 * i + 1]],
                                add=True)

                @pl.when(i < half - 1)
                def _():
                    fire_g(2 * i + 3, 1)

                return carry

            lax.fori_loop(0, half, body, 0)

        @pl.when(c == 0)
        def _():
            edge_loop(a)

        @pl.when(c == 1)
        def _():
            edge_loop(b)

        plsc.subcore_barrier()

        @pl.when(s < NS - 1)
        def _():
            pltpu.sync_copy(acc_sp.at[pl.ds(base, rows_per_tile)],
                            out_hbm.at[c, pl.ds(base, rows_per_tile)])

        @pl.when(s == NS - 1)
        def _():
            pltpu.sync_copy(acc_sp.at[pl.ds(base, last)],
                            out_hbm.at[c, pl.ds(base, last)])

    return agg_kernel


def _tc_matmul(x3, W1):
    n8, _, d = x3.shape
    h = W1.shape[1]

    def body(x_ref, w_ref, h_ref):
        w = w_ref[...]
        for k in range(8):
            h_ref[:, h * k:h * (k + 1)] = jnp.dot(
                x_ref[:, k, :], w, preferred_element_type=jnp.float32)

    return pl.pallas_call(
        body,
        out_shape=jax.ShapeDtypeStruct((n8, 8 * h), jnp.float32),
    )(x3, W1)


def _tc_scale(h_pack, d3):
    n8, hw = h_pack.shape
    h = hw // 8

    def body(h_ref, d_ref, g_ref, s_ref):
        dd = d_ref[...]
        hh = h_ref[...]
        for k in range(8):
            sck = lax.rsqrt(1.0 + dd[0, :, k] + dd[1, :, k])[:, None]
            g_ref[:, h * k:h * (k + 1)] = sck * hh[:, h * k:h * (k + 1)]
            s_ref[:, h * k:h * (k + 1)] = jnp.broadcast_to(sck, (n8, h))

    return pl.pallas_call(
        body,
        out_shape=(
            jax.ShapeDtypeStruct((n8, hw), jnp.float32),
            jax.ShapeDtypeStruct((n8, hw), jnp.float32),
        ),
    )(h_pack, d3)


def _tc_mid(p, g1, s_pack, b1p):
    def body(p_ref, g_ref, s_ref, b_ref, u_ref):
        sc = s_ref[...]
        agg = p_ref[0] + p_ref[1] - g_ref[...]
        u_ref[...] = sc * jnp.maximum(sc * agg + b_ref[...], 0.0)

    return pl.pallas_call(
        body,
        out_shape=jax.ShapeDtypeStruct(g1.shape, jnp.float32),
    )(p, g1, s_pack, b1p)


def _tc_final(q, u, s_pack, W2p, b2p, c):
    n8, hw = u.shape
    cw = 8 * c

    def body(q_ref, u_ref, s_ref, w_ref, b_ref, o_ref):
        t = s_ref[...] * (q_ref[0] + q_ref[1] - u_ref[...])
        z = jnp.dot(t, w_ref[...], preferred_element_type=jnp.float32)
        z = z + b_ref[...]
        zl = jnp.concatenate([z[:, 1:], z[:, :1]], axis=1)
        zr = jnp.concatenate([z[:, cw - 1:], z[:, :cw - 1]], axis=1)
        lane = lax.broadcasted_iota(jnp.int32, (n8, cw), 1)
        partner = jnp.where(lane % 2 == 0, zl, zr)
        m = jnp.maximum(z, partner)
        lse = m + jnp.log(jnp.exp(z - m) + jnp.exp(partner - m))
        o_ref[...] = z - lse

    return pl.pallas_call(
        body,
        out_shape=jax.ShapeDtypeStruct((n8, cw), jnp.float32),
    )(q, u, s_pack, W2p, b2p)


def kernel(x, edge_index, W1, b1, W2, b2):
    n, d = x.shape
    h = W1.shape[1]
    c = W2.shape[1]
    e = edge_index.shape[1]
    n8 = n // 8

    n_acc = ((n + NS * 8) + NS * 8 - 1) // (NS * 8) * (NS * 8)
    p = (e + NS * CHUNK - 1) // (NS * CHUNK)
    p = p + (p % 2)
    e_pad = NS * CHUNK * p

    dst_pad = n + jnp.arange(e_pad - e, dtype=jnp.int32) % (n_acc - n)
    src = jnp.concatenate(
        [edge_index[0], jnp.zeros((e_pad - e,), jnp.int32)]).reshape(
            NS * p, CHUNK)
    dst = jnp.concatenate([edge_index[1], dst_pad]).reshape(NS * p, CHUNK)

    deg_kernel = _make_deg_kernel(n, n_acc, p)
    agg_kernel = _make_agg_kernel(n, n_acc, p, h)

    x3 = x.reshape(n8, 8, d)
    b1p = jnp.tile(b1, 8).reshape(1, 8 * h)
    W2p = jnp.kron(jnp.eye(8, dtype=W2.dtype), W2)
    b2p = jnp.tile(b2, 8).reshape(1, 8 * c)

    h_pack = _tc_matmul(x3, W1)
    degp = deg_kernel(dst)
    g1, s_pack = _tc_scale(h_pack, degp.reshape(NC, n8, 8))
    pagg = agg_kernel(g1.reshape(n, h), src, dst)
    u = _tc_mid(pagg.reshape(NC, n8, 8 * h), g1, s_pack, b1p)
    q = agg_kernel(u.reshape(n, h), src, dst)
    out = _tc_final(q.reshape(NC, n8, 8 * h), u, s_pack, W2p, b2p, c)
    return out.reshape(n, c)


# speedup vs baseline: 1.1289x; 1.0271x over previous
import functools

import jax
import jax.numpy as jnp
from jax import lax
from jax.experimental import pallas as pl
from jax.experimental.pallas import tpu as pltpu
from jax.experimental.pallas import tpu_sc as plsc

NC = 2
NS = 16
CHUNK = 128
CORE0_FRAC = 0.65


def _mesh():
    return plsc.VectorSubcoreMesh(core_axis_name="c", subcore_axis_name="s")


def _chunk_split(p):
    a = min(p - 8, max(8, int(round(p * CORE0_FRAC / 4)) * 4))
    return a, p - a


def _make_deg_kernel(n, n_acc, p):
    rows_per_tile = n_acc // NS
    last = n - (NS - 1) * rows_per_tile
    a, b = _chunk_split(p)

    @functools.partial(
        pl.kernel,
        out_type=jax.ShapeDtypeStruct((NC, n), jnp.float32),
        mesh=_mesh(),
        scratch_types=[
            pltpu.VMEM((max(a, b), CHUNK), jnp.int32),
            pltpu.VMEM((CHUNK,), jnp.float32),
            pltpu.VMEM((CHUNK,), jnp.float32),
            pltpu.VMEM_SHARED((n_acc,), jnp.float32),
            pltpu.SemaphoreType.DMA,
        ],
        compiler_params=pltpu.CompilerParams(use_tc_tiling_on_sc=False),
    )
    def deg_kernel(dst_hbm, out_hbm, idx_v, ones_v, zeros_v, acc_sp, sem):
        c = lax.axis_index("c")
        s = lax.axis_index("s")
        for i in range(CHUNK // 16):
            ones_v[pl.ds(16 * i, 16)] = jnp.ones((16,), jnp.float32)
            zeros_v[pl.ds(16 * i, 16)] = jnp.zeros((16,), jnp.float32)
        base = s * rows_per_tile
        for k in range(rows_per_tile // CHUNK):
            pltpu.sync_copy(zeros_v, acc_sp.at[pl.ds(base + k * CHUNK, CHUNK)])

        def scatter_loop(cnt):
            depth = 8

            def wait_s():
                pltpu.make_async_copy(out_hbm.at[0, pl.ds(0, CHUNK)],
                                      ones_v, sem).wait()

            def body(j, carry):
                pltpu.async_copy(ones_v, acc_sp.at[idx_v.at[j]], sem,
                                 add=True)

                @pl.when(j >= depth)
                def _():
                    wait_s()

                return carry

            lax.fori_loop(0, cnt, body, 0)
            for _ in range(depth):
                wait_s()

        @pl.when(c == 0)
        def _():
            pltpu.sync_copy(dst_hbm.at[pl.ds(s * a, a)],
                            idx_v.at[pl.ds(0, a)])

        @pl.when(c == 1)
        def _():
            pltpu.sync_copy(dst_hbm.at[pl.ds(NS * a + s * b, b)],
                            idx_v.at[pl.ds(0, b)])

        plsc.subcore_barrier()

        @pl.when(c == 0)
        def _():
            scatter_loop(a)

        @pl.when(c == 1)
        def _():
            scatter_loop(b)

        plsc.subcore_barrier()

        @pl.when(s < NS - 1)
        def _():
            pltpu.sync_copy(acc_sp.at[pl.ds(base, rows_per_tile)],
                            out_hbm.at[c, pl.ds(base, rows_per_tile)])

        @pl.when(s == NS - 1)
        def _():
            pltpu.sync_copy(acc_sp.at[pl.ds(base, last)],
                            out_hbm.at[c, pl.ds(base, last)])

    return deg_kernel


def _make_agg_kernel(n, n_acc, p, width):
    rows_per_tile = n_acc // NS
    last = n - (NS - 1) * rows_per_tile
    a, b = _chunk_split(p)

    @functools.partial(
        pl.kernel,
        out_type=jax.ShapeDtypeStruct((NC, n, width), jnp.float32),
        mesh=_mesh(),
        scratch_types=[
            pltpu.VMEM((max(a, b), CHUNK), jnp.int32),
            pltpu.VMEM((max(a, b), CHUNK), jnp.int32),
            pltpu.VMEM((4, CHUNK, width), jnp.float32),
            pltpu.VMEM_SHARED((n_acc, width), jnp.float32),
            pltpu.SemaphoreType.DMA,
            pltpu.SemaphoreType.DMA,
        ],
        compiler_params=pltpu.CompilerParams(use_tc_tiling_on_sc=False),
    )
    def agg_kernel(g_hbm, src_hbm, dst_hbm, out_hbm, si_v, di_v, msg_v,
                   acc_sp, sem_g, sem_s):
        c = lax.axis_index("c")
        s = lax.axis_index("s")
        base = s * rows_per_tile
        @pl.when(s < NS - 1)
        def _():
            pltpu.sync_copy(g_hbm.at[pl.ds(base, rows_per_tile)],
                            acc_sp.at[pl.ds(base, rows_per_tile)])

        @pl.when(s == NS - 1)
        def _():
            pltpu.sync_copy(g_hbm.at[pl.ds(base, last)],
                            acc_sp.at[pl.ds(base, last)])

        @pl.when(c == 0)
        def _():
            pltpu.sync_copy(src_hbm.at[pl.ds(s * a, a)],
                            si_v.at[pl.ds(0, a)])
            pltpu.sync_copy(dst_hbm.at[pl.ds(s * a, a)],
                            di_v.at[pl.ds(0, a)])

        @pl.when(c == 1)
        def _():
            pltpu.sync_copy(src_hbm.at[pl.ds(NS * a + s * b, b)],
                            si_v.at[pl.ds(0, b)])
            pltpu.sync_copy(dst_hbm.at[pl.ds(NS * a + s * b, b)],
                            di_v.at[pl.ds(0, b)])

        plsc.subcore_barrier()

        def edge_loop(cnt):
            def fire_g(j, slot):
                pltpu.async_copy(g_hbm.at[si_v.at[j]], msg_v.at[slot],
                                 sem_g)

            def wait_g():
                pltpu.make_async_copy(g_hbm.at[pl.ds(0, CHUNK)],
                                      msg_v.at[0], sem_g).wait()

            for b in range(4):
                fire_g(b, b)

            def body(i, carry):
                for b in range(4):
                    j = 4 * i + b
                    wait_g()
                    pltpu.sync_copy(msg_v.at[b], acc_sp.at[di_v.at[j]],
                                    add=True)

                    @pl.when(j + 4 < cnt)
                    def _():
                        fire_g(j + 4, b)

                return carry

            lax.fori_loop(0, cnt // 4, body, 0)

        @pl.when(c == 0)
        def _():
            edge_loop(a)

        @pl.when(c == 1)
        def _():
            edge_loop(b)

        plsc.subcore_barrier()

        @pl.when(s < NS - 1)
        def _():
            pltpu.sync_copy(acc_sp.at[pl.ds(base, rows_per_tile)],
                            out_hbm.at[c, pl.ds(base, rows_per_tile)])

        @pl.when(s == NS - 1)
        def _():
            pltpu.sync_copy(acc_sp.at[pl.ds(base, last)],
                            out_hbm.at[c, pl.ds(base, last)])

    return agg_kernel


def _tc_matmul(x3, W1):
    n8, _, d = x3.shape
    h = W1.shape[1]

    def body(x_ref, w_ref, h_ref):
        w = w_ref[...]
        for k in range(8):
            h_ref[:, h * k:h * (k + 1)] = jnp.dot(
                x_ref[:, k, :], w, preferred_element_type=jnp.float32)

    return pl.pallas_call(
        body,
        out_shape=jax.ShapeDtypeStruct((n8, 8 * h), jnp.float32),
    )(x3, W1)


def _tc_scale(h_pack, d3):
    n8, hw = h_pack.shape
    h = hw // 8

    def body(h_ref, d_ref, g_ref, s_ref):
        dd = d_ref[...]
        hh = h_ref[...]
        for k in range(8):
            sck = lax.rsqrt(1.0 + dd[0, :, k] + dd[1, :, k])[:, None]
            g_ref[:, h * k:h * (k + 1)] = sck * hh[:, h * k:h * (k + 1)]
            s_ref[:, h * k:h * (k + 1)] = jnp.broadcast_to(sck, (n8, h))

    return pl.pallas_call(
        body,
        out_shape=(
            jax.ShapeDtypeStruct((n8, hw), jnp.float32),
            jax.ShapeDtypeStruct((n8, hw), jnp.float32),
        ),
    )(h_pack, d3)


def _tc_mid(p, g1, s_pack, b1p):
    def body(p_ref, g_ref, s_ref, b_ref, u_ref):
        sc = s_ref[...]
        agg = p_ref[0] + p_ref[1] - g_ref[...]
        u_ref[...] = sc * jnp.maximum(sc * agg + b_ref[...], 0.0)

    return pl.pallas_call(
        body,
        out_shape=jax.ShapeDtypeStruct(g1.shape, jnp.float32),
    )(p, g1, s_pack, b1p)


def _tc_final(q, u, s_pack, W2p, b2p, c):
    n8, hw = u.shape
    cw = 8 * c

    def body(q_ref, u_ref, s_ref, w_ref, b_ref, o_ref):
        t = s_ref[...] * (q_ref[0] + q_ref[1] - u_ref[...])
        z = jnp.dot(t, w_ref[...], preferred_element_type=jnp.float32)
        z = z + b_ref[...]
        zl = jnp.concatenate([z[:, 1:], z[:, :1]], axis=1)
        zr = jnp.concatenate([z[:, cw - 1:], z[:, :cw - 1]], axis=1)
        lane = lax.broadcasted_iota(jnp.int32, (n8, cw), 1)
        partner = jnp.where(lane % 2 == 0, zl, zr)
        m = jnp.maximum(z, partner)
        lse = m + jnp.log(jnp.exp(z - m) + jnp.exp(partner - m))
        o_ref[...] = z - lse

    return pl.pallas_call(
        body,
        out_shape=jax.ShapeDtypeStruct((n8, cw), jnp.float32),
    )(q, u, s_pack, W2p, b2p)


def kernel(x, edge_index, W1, b1, W2, b2):
    n, d = x.shape
    h = W1.shape[1]
    c = W2.shape[1]
    e = edge_index.shape[1]
    n8 = n // 8

    n_acc = ((n + NS * 8) + NS * 8 - 1) // (NS * 8) * (NS * 8)
    p = (e + NS * CHUNK - 1) // (NS * CHUNK)
    p = p + (p % 2)
    e_pad = NS * CHUNK * p

    dst_pad = n + jnp.arange(e_pad - e, dtype=jnp.int32) % (n_acc - n)
    src = jnp.concatenate(
        [edge_index[0], jnp.zeros((e_pad - e,), jnp.int32)]).reshape(
            NS * p, CHUNK)
    dst = jnp.concatenate([edge_index[1], dst_pad]).reshape(NS * p, CHUNK)

    deg_kernel = _make_deg_kernel(n, n_acc, p)
    agg_kernel = _make_agg_kernel(n, n_acc, p, h)

    x3 = x.reshape(n8, 8, d)
    b1p = jnp.tile(b1, 8).reshape(1, 8 * h)
    W2p = jnp.kron(jnp.eye(8, dtype=W2.dtype), W2)
    b2p = jnp.tile(b2, 8).reshape(1, 8 * c)

    h_pack = _tc_matmul(x3, W1)
    degp = deg_kernel(dst)
    g1, s_pack = _tc_scale(h_pack, degp.reshape(NC, n8, 8))
    pagg = agg_kernel(g1.reshape(n, h), src, dst)
    u = _tc_mid(pagg.reshape(NC, n8, 8 * h), g1, s_pack, b1p)
    q = agg_kernel(u.reshape(n, h), src, dst)
    out = _tc_final(q.reshape(NC, n8, 8 * h), u, s_pack, W2p, b2p, c)
    return out.reshape(n, c)
